# baseline (speedup 1.0000x reference)
import functools

import jax
import jax.numpy as jnp
from jax import lax
from jax.experimental import pallas as pl
from jax.experimental.pallas import tpu as pltpu
from jax.experimental.pallas import tpu_sc as plsc

N = 10000
D_IN = 256
E = 160000

NC = 2
NS = 16
DH = D_IN // NC
EPW = E // NS
CH = 80
NCH = EPW // CH
K = 4
RPW = 624
RPW_LAST = N - (NS - 1) * RPW


def _sc_aggregate(srcA, srcB, dst, x3, xr):
    mesh = plsc.VectorSubcoreMesh(core_axis_name="c", subcore_axis_name="s",
                                  num_cores=NC, num_subcores=NS)

    scratch = dict(
        acc=pltpu.VMEM_SHARED((N, DH), jnp.float32),
        isem=pltpu.SemaphoreType.DMA((K,)),
        jsem=pltpu.SemaphoreType.DMA((K,)),
        gsem=pltpu.SemaphoreType.DMA((K,)),
        ssem=pltpu.SemaphoreType.DMA((K,)),
    )
    for b in range(K):
        scratch[f"sidx{b}"] = pltpu.VMEM((CH,), jnp.int32)
        scratch[f"didx{b}"] = pltpu.VMEM((CH,), jnp.int32)
        scratch[f"rows{b}"] = pltpu.VMEM((CH, DH), jnp.float32)

    @functools.partial(
        pl.kernel,
        out_type=(jax.ShapeDtypeStruct((N, DH), jnp.float32),
                  jax.ShapeDtypeStruct((N, DH), jnp.float32)),
        mesh=mesh,
        scratch_types=scratch,
    )
    def agg(srcA_hbm, srcB_hbm, dst_hbm, x3_hbm, xr_hbm, oa_hbm, ob_hbm,
            **scr):
        c = lax.axis_index("c")
        s = lax.axis_index("s")
        sidx = [scr[f"sidx{b}"] for b in range(K)]
        didx = [scr[f"didx{b}"] for b in range(K)]
        rows = [scr[f"rows{b}"] for b in range(K)]
        acc = scr["acc"]
        isem, jsem = scr["isem"], scr["jsem"]
        gsem, ssem = scr["gsem"], scr["ssem"]

        @pl.when(s < NS - 1)
        def _():
            pltpu.sync_copy(x3_hbm.at[pl.ds(s * RPW, RPW), c],
                            acc.at[pl.ds(s * RPW, RPW)])

        @pl.when(s == NS - 1)
        def _():
            pltpu.sync_copy(x3_hbm.at[pl.ds((NS - 1) * RPW, RPW_LAST), c],
                            acc.at[pl.ds((NS - 1) * RPW, RPW_LAST)])

        plsc.subcore_barrier()

        base = s * EPW

        def pipeline(src_hbm, x_hbm):
            def wait_scatter(b):
                pltpu.make_async_copy(rows[b], acc.at[didx[b]],
                                      ssem.at[b]).wait()

            def f1(f, b):
                pltpu.async_copy(src_hbm.at[pl.ds(base + f * CH, CH)],
                                 sidx[b], isem.at[b])
                pltpu.async_copy(dst_hbm.at[pl.ds(base + f * CH, CH)],
                                 didx[b], jsem.at[b])

            def f1_guarded(f, b):
                @pl.when(f >= K)
                def _():
                    wait_scatter(b)

                f1(f, b)

            def f2(b):
                pltpu.make_async_copy(src_hbm.at[pl.ds(base, CH)],
                                      sidx[b], isem.at[b]).wait()
                pltpu.async_copy(x_hbm.at[sidx[b]], rows[b], gsem.at[b])

            def s3(b):
                pltpu.make_async_copy(dst_hbm.at[pl.ds(base, CH)],
                                      didx[b], jsem.at[b]).wait()
                pltpu.make_async_copy(x_hbm.at[sidx[b]], rows[b],
                                      gsem.at[b]).wait()
                pltpu.async_copy(rows[b], acc.at[didx[b]], ssem.at[b],
                                 add=True)

            f1(0, 0)
            f1(1, 1)
            f2(0)

            def body(q, carry):
                g0 = q * K
                for b in range(K):
                    g = g0 + b

                    @pl.when(g + 2 < NCH)
                    def _():
                        f1_guarded(g + 2, (b + 2) % K)

                    f2((b + 1) % K)
                    s3(b)
                return carry

            lax.fori_loop(0, NCH // K, body, 0)

            s3(0)
            for b in range(K):
                wait_scatter(b)

        @pl.when(c == 0)
        def _():
            pipeline(srcA_hbm, xr_hbm)

        @pl.when(c == 1)
        def _():
            pipeline(srcB_hbm, xr_hbm)

        plsc.subcore_barrier()

        def write_to(o_hbm):
            @pl.when(s < NS - 1)
            def _():
                pltpu.sync_copy(acc.at[pl.ds(s * RPW, RPW)],
                                o_hbm.at[pl.ds(s * RPW, RPW)])

            @pl.when(s == NS - 1)
            def _():
                pltpu.sync_copy(acc.at[pl.ds((NS - 1) * RPW, RPW_LAST)],
                                o_hbm.at[pl.ds((NS - 1) * RPW, RPW_LAST)])

        @pl.when(c == 0)
        def _():
            write_to(oa_hbm)

        @pl.when(c == 1)
        def _():
            write_to(ob_hbm)

    return agg(srcA, srcB, dst, x3, xr)


ROW_BLK = 2000


def _mlp_kernel(ha_ref, hb_ref, w1a_ref, w1b_ref, b1_ref, w2_ref, b2_ref,
                w3_ref, b3_ref, o_ref):
    t = jnp.dot(ha_ref[...], w1a_ref[...],
                preferred_element_type=jnp.float32)
    t += jnp.dot(hb_ref[...], w1b_ref[...],
                 preferred_element_type=jnp.float32)
    t = jnp.maximum(t + b1_ref[...], 0.0)
    t = jnp.dot(t, w2_ref[...], preferred_element_type=jnp.float32)
    t = jnp.maximum(t + b2_ref[...], 0.0)
    t = jnp.dot(t, w3_ref[...], preferred_element_type=jnp.float32)
    o_ref[...] = t + b3_ref[...]


def _mlp(ha, hb, W1, b1, W2, b2, W3, b3):
    d_hid = W1.shape[1]
    d_out = W3.shape[1]
    full = lambda r, c_: pl.BlockSpec((r, c_), lambda i: (0, 0))
    return pl.pallas_call(
        _mlp_kernel,
        grid=(N // ROW_BLK,),
        in_specs=[
            pl.BlockSpec((ROW_BLK, DH), lambda i: (i, 0)),
            pl.BlockSpec((ROW_BLK, DH), lambda i: (i, 0)),
            full(DH, d_hid),
            full(DH, d_hid),
            full(1, d_hid),
            full(d_hid, d_hid),
            full(1, d_hid),
            full(d_hid, d_out),
            full(1, d_out),
        ],
        out_specs=pl.BlockSpec((ROW_BLK, d_out), lambda i: (i, 0)),
        out_shape=jax.ShapeDtypeStruct((N, d_out), jnp.float32),
    )(ha, hb, W1[:DH], W1[DH:], b1.reshape(1, -1), W2, b2.reshape(1, -1),
      W3, b3.reshape(1, -1))


def kernel(x, edge_index, W1, b1, W2, b2, W3, b3):
    src = edge_index[0]
    dst = edge_index[1]
    srcA = src * 2
    srcB = srcA + 1
    x3 = x.reshape(N, NC, DH)
    xr = x.reshape(N * NC, DH)
    ha, hb = _sc_aggregate(srcA, srcB, dst, x3, xr)
    return _mlp(ha, hb, W1, b1, W2, b2, W3, b3)


# speedup vs baseline: 1.0874x; 1.0874x over previous
import functools

import jax
import jax.numpy as jnp
from jax import lax
from jax.experimental import pallas as pl
from jax.experimental.pallas import tpu as pltpu
from jax.experimental.pallas import tpu_sc as plsc

N = 10000
D_IN = 256
E = 160000

NC = 2
NS = 16
DH = D_IN // NC
EPW = E // NS
CH = 80
NCH = EPW // CH
KI = 8
KR = 4
RPW = 624
RPW_LAST = N - (NS - 1) * RPW


def _sc_aggregate(src, dst, xa, xb):
    mesh = plsc.VectorSubcoreMesh(core_axis_name="c", subcore_axis_name="s",
                                  num_cores=NC, num_subcores=NS)

    scratch = dict(
        acc=pltpu.VMEM_SHARED((N, DH), jnp.float32),
        isem=pltpu.SemaphoreType.DMA((KI,)),
        jsem=pltpu.SemaphoreType.DMA((KI,)),
        gsem=pltpu.SemaphoreType.DMA((KR,)),
        ssem=pltpu.SemaphoreType.DMA((KR,)),
    )
    for b in range(KI):
        scratch[f"sidx{b}"] = pltpu.VMEM((CH,), jnp.int32)
        scratch[f"didx{b}"] = pltpu.VMEM((CH,), jnp.int32)
    for b in range(KR):
        scratch[f"rows{b}"] = pltpu.VMEM((CH, DH), jnp.float32)

    @functools.partial(
        pl.kernel,
        out_type=(jax.ShapeDtypeStruct((N, DH), jnp.float32),
                  jax.ShapeDtypeStruct((N, DH), jnp.float32)),
        mesh=mesh,
        scratch_types=scratch,
    )
    def agg(src_hbm, dst_hbm, xa_hbm, xb_hbm, oa_hbm, ob_hbm, **scr):
        c = lax.axis_index("c")
        s = lax.axis_index("s")
        sidx = [scr[f"sidx{b}"] for b in range(KI)]
        didx = [scr[f"didx{b}"] for b in range(KI)]
        rows = [scr[f"rows{b}"] for b in range(KR)]
        acc = scr["acc"]
        isem, jsem = scr["isem"], scr["jsem"]
        gsem, ssem = scr["gsem"], scr["ssem"]

        def init_from(x_hbm):
            @pl.when(s < NS - 1)
            def _():
                pltpu.sync_copy(x_hbm.at[pl.ds(s * RPW, RPW)],
                                acc.at[pl.ds(s * RPW, RPW)])

            @pl.when(s == NS - 1)
            def _():
                pltpu.sync_copy(x_hbm.at[pl.ds((NS - 1) * RPW, RPW_LAST)],
                                acc.at[pl.ds((NS - 1) * RPW, RPW_LAST)])

        @pl.when(c == 0)
        def _():
            init_from(xa_hbm)

        @pl.when(c == 1)
        def _():
            init_from(xb_hbm)

        plsc.subcore_barrier()

        base = s * EPW

        def pipeline(x_hbm):
            def wait_scatter(br):
                pltpu.make_async_copy(rows[br], acc.at[didx[0]],
                                      ssem.at[br]).wait()

            def f1(f, bi):
                pltpu.async_copy(src_hbm.at[pl.ds(base + f * CH, CH)],
                                 sidx[bi], isem.at[bi])
                pltpu.async_copy(dst_hbm.at[pl.ds(base + f * CH, CH)],
                                 didx[bi], jsem.at[bi])

            def f2(f, bi, br):
                pltpu.make_async_copy(src_hbm.at[pl.ds(base, CH)],
                                      sidx[bi], isem.at[bi]).wait()

                @pl.when(f >= KR)
                def _():
                    wait_scatter(br)

                pltpu.async_copy(x_hbm.at[sidx[bi]], rows[br],
                                 gsem.at[br])

            def s3(bi, br):
                pltpu.make_async_copy(dst_hbm.at[pl.ds(base, CH)],
                                      didx[bi], jsem.at[bi]).wait()
                pltpu.make_async_copy(x_hbm.at[sidx[bi]], rows[br],
                                      gsem.at[br]).wait()
                pltpu.async_copy(rows[br], acc.at[didx[bi]], ssem.at[br],
                                 add=True)

            for f in range(4):
                f1(f, f)
            f2(0, 0, 0)
            f2(1, 1, 1)

            def body(q, carry):
                g0 = q * KI
                for b in range(KI):
                    g = g0 + b
                    f1(g + 4, (b + 4) % KI)
                    f2(g + 2, (b + 2) % KI, (b + 2) % KR)
                    s3(b, b % KR)
                return carry

            lax.fori_loop(0, NCH // KI, body, 0)

            g = (NCH // KI) * KI
            f1(g + 4, (g + 4) % KI)
            f2(g + 2, (g + 2) % KI, (g + 2) % KR)
            s3(g % KI, g % KR)
            f2(g + 3, (g + 3) % KI, (g + 3) % KR)
            s3((g + 1) % KI, (g + 1) % KR)
            f2(g + 4, (g + 4) % KI, (g + 4) % KR)
            s3((g + 2) % KI, (g + 2) % KR)
            s3((g + 3) % KI, (g + 3) % KR)
            s3((g + 4) % KI, (g + 4) % KR)
            for br in [(g + 1) % KR, (g + 2) % KR, (g + 3) % KR,
                       (g + 4) % KR]:
                wait_scatter(br)

        @pl.when(c == 0)
        def _():
            pipeline(xa_hbm)

        @pl.when(c == 1)
        def _():
            pipeline(xb_hbm)

        plsc.subcore_barrier()

        def write_to(o_hbm):
            @pl.when(s < NS - 1)
            def _():
                pltpu.sync_copy(acc.at[pl.ds(s * RPW, RPW)],
                                o_hbm.at[pl.ds(s * RPW, RPW)])

            @pl.when(s == NS - 1)
            def _():
                pltpu.sync_copy(acc.at[pl.ds((NS - 1) * RPW, RPW_LAST)],
                                o_hbm.at[pl.ds((NS - 1) * RPW, RPW_LAST)])

        @pl.when(c == 0)
        def _():
            write_to(oa_hbm)

        @pl.when(c == 1)
        def _():
            write_to(ob_hbm)

    return agg(src, dst, xa, xb)


ROW_BLK = 2000


def _mlp_kernel(ha_ref, hb_ref, w1a_ref, w1b_ref, b1_ref, w2_ref, b2_ref,
                w3_ref, b3_ref, o_ref):
    t = jnp.dot(ha_ref[...], w1a_ref[...],
                preferred_element_type=jnp.float32)
    t += jnp.dot(hb_ref[...], w1b_ref[...],
                 preferred_element_type=jnp.float32)
    t = jnp.maximum(t + b1_ref[...], 0.0)
    t = jnp.dot(t, w2_ref[...], preferred_element_type=jnp.float32)
    t = jnp.maximum(t + b2_ref[...], 0.0)
    t = jnp.dot(t, w3_ref[...], preferred_element_type=jnp.float32)
    o_ref[...] = t + b3_ref[...]


def _mlp(ha, hb, W1, b1, W2, b2, W3, b3):
    d_hid = W1.shape[1]
    d_out = W3.shape[1]
    full = lambda r, c_: pl.BlockSpec((r, c_), lambda i: (0, 0))
    return pl.pallas_call(
        _mlp_kernel,
        grid=(N // ROW_BLK,),
        in_specs=[
            pl.BlockSpec((ROW_BLK, DH), lambda i: (i, 0)),
            pl.BlockSpec((ROW_BLK, DH), lambda i: (i, 0)),
            full(DH, d_hid),
            full(DH, d_hid),
            full(1, d_hid),
            full(d_hid, d_hid),
            full(1, d_hid),
            full(d_hid, d_out),
            full(1, d_out),
        ],
        out_specs=pl.BlockSpec((ROW_BLK, d_out), lambda i: (i, 0)),
        out_shape=jax.ShapeDtypeStruct((N, d_out), jnp.float32),
    )(ha, hb, W1[:DH], W1[DH:], b1.reshape(1, -1), W2, b2.reshape(1, -1),
      W3, b3.reshape(1, -1))


def kernel(x, edge_index, W1, b1, W2, b2, W3, b3):
    src = edge_index[0]
    dst = edge_index[1]
    xa = x[:, :DH]
    xb = x[:, DH:]
    ha, hb = _sc_aggregate(src, dst, xa, xb)
    return _mlp(ha, hb, W1, b1, W2, b2, W3, b3)


# speedup vs baseline: 1.1179x; 1.0281x over previous
import functools

import jax
import jax.numpy as jnp
from jax import lax
from jax.experimental import pallas as pl
from jax.experimental.pallas import tpu as pltpu
from jax.experimental.pallas import tpu_sc as plsc

N = 10000
D_IN = 256
E = 160000

NC = 2
NS = 16
DH = D_IN // NC
EPW = E // NS
CH = 80
NCH = EPW // CH
K = 4
RPW = 624
RPW_LAST = N - (NS - 1) * RPW


def _sc_aggregate(src, dst, xa, xb):
    mesh = plsc.VectorSubcoreMesh(core_axis_name="c", subcore_axis_name="s",
                                  num_cores=NC, num_subcores=NS)

    scratch = dict(
        acc=pltpu.VMEM_SHARED((N, DH), jnp.float32),
        isem=pltpu.SemaphoreType.DMA((K,)),
        jsem=pltpu.SemaphoreType.DMA((K,)),
        gsem=pltpu.SemaphoreType.DMA((K,)),
        ssem=pltpu.SemaphoreType.DMA((K,)),
    )
    for b in range(K):
        scratch[f"sidx{b}"] = pltpu.VMEM((CH,), jnp.int32)
        scratch[f"didx{b}"] = pltpu.VMEM((CH,), jnp.int32)
        scratch[f"rows{b}"] = pltpu.VMEM((CH, DH), jnp.float32)

    @functools.partial(
        pl.kernel,
        out_type=(jax.ShapeDtypeStruct((N, DH), jnp.float32),
                  jax.ShapeDtypeStruct((N, DH), jnp.float32)),
        mesh=mesh,
        scratch_types=scratch,
    )
    def agg(src_hbm, dst_hbm, xa_hbm, xb_hbm, oa_hbm, ob_hbm, **scr):
        c = lax.axis_index("c")
        s = lax.axis_index("s")
        sidx = [scr[f"sidx{b}"] for b in range(K)]
        didx = [scr[f"didx{b}"] for b in range(K)]
        rows = [scr[f"rows{b}"] for b in range(K)]
        acc = scr["acc"]
        isem, jsem = scr["isem"], scr["jsem"]
        gsem, ssem = scr["gsem"], scr["ssem"]

        def init_from(x_hbm):
            @pl.when(s < NS - 1)
            def _():
                pltpu.sync_copy(x_hbm.at[pl.ds(s * RPW, RPW)],
                                acc.at[pl.ds(s * RPW, RPW)])

            @pl.when(s == NS - 1)
            def _():
                pltpu.sync_copy(x_hbm.at[pl.ds((NS - 1) * RPW, RPW_LAST)],
                                acc.at[pl.ds((NS - 1) * RPW, RPW_LAST)])

        base = s * EPW

        def pipeline(x_hbm):
            def wait_scatter(b):
                pltpu.make_async_copy(rows[b], acc.at[didx[b]],
                                      ssem.at[b]).wait()

            def f1(f, b):
                pltpu.async_copy(src_hbm.at[pl.ds(base + f * CH, CH)],
                                 sidx[b], isem.at[b])
                pltpu.async_copy(dst_hbm.at[pl.ds(base + f * CH, CH)],
                                 didx[b], jsem.at[b])

            def f1_guarded(f, b):
                @pl.when(f >= K)
                def _():
                    wait_scatter(b)

                f1(f, b)

            def f2(b):
                pltpu.make_async_copy(src_hbm.at[pl.ds(base, CH)],
                                      sidx[b], isem.at[b]).wait()
                pltpu.async_copy(x_hbm.at[sidx[b]], rows[b], gsem.at[b])

            def s3(b):
                pltpu.make_async_copy(dst_hbm.at[pl.ds(base, CH)],
                                      didx[b], jsem.at[b]).wait()
                pltpu.make_async_copy(x_hbm.at[sidx[b]], rows[b],
                                      gsem.at[b]).wait()
                pltpu.async_copy(rows[b], acc.at[didx[b]], ssem.at[b],
                                 add=True)

            f1(0, 0)
            f1(1, 1)
            f2(0)
            init_from(x_hbm)
            plsc.subcore_barrier()

            def body(q, carry):
                g0 = q * K
                for b in range(K):
                    g = g0 + b

                    @pl.when(g + 2 < NCH)
                    def _():
                        f1_guarded(g + 2, (b + 2) % K)

                    f2((b + 1) % K)
                    s3(b)
                return carry

            lax.fori_loop(0, NCH // K, body, 0)

            s3(0)
            for b in range(K):
                wait_scatter(b)

        @pl.when(c == 0)
        def _():
            pipeline(xa_hbm)

        @pl.when(c == 1)
        def _():
            pipeline(xb_hbm)

        plsc.subcore_barrier()

        def write_to(o_hbm):
            @pl.when(s < NS - 1)
            def _():
                pltpu.sync_copy(acc.at[pl.ds(s * RPW, RPW)],
                                o_hbm.at[pl.ds(s * RPW, RPW)])

            @pl.when(s == NS - 1)
            def _():
                pltpu.sync_copy(acc.at[pl.ds((NS - 1) * RPW, RPW_LAST)],
                                o_hbm.at[pl.ds((NS - 1) * RPW, RPW_LAST)])

        @pl.when(c == 0)
        def _():
            write_to(oa_hbm)

        @pl.when(c == 1)
        def _():
            write_to(ob_hbm)

    return agg(src, dst, xa, xb)


ROW_BLK = 2000


def _mlp_kernel(ha_ref, hb_ref, w1a_ref, w1b_ref, b1_ref, w2_ref, b2_ref,
                w3_ref, b3_ref, o_ref):
    t = jnp.dot(ha_ref[...], w1a_ref[...],
                preferred_element_type=jnp.float32)
    t += jnp.dot(hb_ref[...], w1b_ref[...],
                 preferred_element_type=jnp.float32)
    t = jnp.maximum(t + b1_ref[...], 0.0)
    t = jnp.dot(t, w2_ref[...], preferred_element_type=jnp.float32)
    t = jnp.maximum(t + b2_ref[...], 0.0)
    t = jnp.dot(t, w3_ref[...], preferred_element_type=jnp.float32)
    o_ref[...] = t + b3_ref[...]


def _mlp(ha, hb, W1, b1, W2, b2, W3, b3):
    d_hid = W1.shape[1]
    d_out = W3.shape[1]
    full = lambda r, c_: pl.BlockSpec((r, c_), lambda i: (0, 0))
    return pl.pallas_call(
        _mlp_kernel,
        grid=(N // ROW_BLK,),
        in_specs=[
            pl.BlockSpec((ROW_BLK, DH), lambda i: (i, 0)),
            pl.BlockSpec((ROW_BLK, DH), lambda i: (i, 0)),
            full(DH, d_hid),
            full(DH, d_hid),
            full(1, d_hid),
            full(d_hid, d_hid),
            full(1, d_hid),
            full(d_hid, d_out),
            full(1, d_out),
        ],
        out_specs=pl.BlockSpec((ROW_BLK, d_out), lambda i: (i, 0)),
        out_shape=jax.ShapeDtypeStruct((N, d_out), jnp.float32),
    )(ha, hb, W1[:DH], W1[DH:], b1.reshape(1, -1), W2, b2.reshape(1, -1),
      W3, b3.reshape(1, -1))


def kernel(x, edge_index, W1, b1, W2, b2, W3, b3):
    src = edge_index[0]
    dst = edge_index[1]
    xa = x[:, :DH]
    xb = x[:, DH:]
    ha, hb = _sc_aggregate(src, dst, xa, xb)
    return _mlp(ha, hb, W1, b1, W2, b2, W3, b3)
